# Initial kernel scaffold; baseline (speedup 1.0000x reference)
import jax
import jax.numpy as jnp
from jax.experimental import pallas as pl


def kernel(x):
    raise NotImplementedError("write your pallas kernel here")


# speedup vs baseline: 1.4879x; 1.4879x over previous
import jax
import jax.numpy as jnp
from jax.experimental import pallas as pl
from jax.experimental.pallas import tpu as pltpu


def _softmax_block_kernel(x_ref, lp_ref, p_ref):
    x = x_ref[...]
    m = jnp.max(x, axis=-1, keepdims=True)
    e = jnp.exp(x - m)
    s = jnp.sum(e, axis=-1, keepdims=True)
    lp_ref[...] = x - (m + jnp.log(s))
    p_ref[...] = e * (1.0 / s)


def kernel(x):
    B, H, N = x.shape
    rows = B * H
    xf = x.reshape(rows, N)
    R = 16
    grid = (rows // R,)
    lp, p = pl.pallas_call(
        _softmax_block_kernel,
        grid=grid,
        in_specs=[pl.BlockSpec((R, N), lambda i: (i, 0))],
        out_specs=[
            pl.BlockSpec((R, N), lambda i: (i, 0)),
            pl.BlockSpec((R, N), lambda i: (i, 0)),
        ],
        out_shape=[
            jax.ShapeDtypeStruct((rows, N), x.dtype),
            jax.ShapeDtypeStruct((rows, N), x.dtype),
        ],
        compiler_params=pltpu.CompilerParams(
            dimension_semantics=("arbitrary",),
        ),
    )(xf)
    return lp.reshape(B, H, N), p.reshape(B, H, N)


# speedup vs baseline: 1.6028x; 1.0772x over previous
import jax
import jax.numpy as jnp
from jax.experimental import pallas as pl
from jax.experimental.pallas import tpu as pltpu


def _softmax_block_kernel(x_ref, lp_ref, p_ref):
    x = x_ref[...]
    m = jnp.max(x, axis=-1, keepdims=True)
    e = jnp.exp(x - m)
    s = jnp.sum(e, axis=-1, keepdims=True)
    lp_ref[...] = x - (m + jnp.log(s))
    p_ref[...] = e * (1.0 / s)


def kernel(x):
    B, H, N = x.shape
    rows = B * H
    xf = x.reshape(rows, N)
    R = 32
    grid = (rows // R,)
    lp, p = pl.pallas_call(
        _softmax_block_kernel,
        grid=grid,
        in_specs=[pl.BlockSpec((R, N), lambda i: (i, 0))],
        out_specs=[
            pl.BlockSpec((R, N), lambda i: (i, 0)),
            pl.BlockSpec((R, N), lambda i: (i, 0)),
        ],
        out_shape=[
            jax.ShapeDtypeStruct((rows, N), x.dtype),
            jax.ShapeDtypeStruct((rows, N), x.dtype),
        ],
        compiler_params=pltpu.CompilerParams(
            dimension_semantics=("arbitrary",),
        ),
    )(xf)
    return lp.reshape(B, H, N), p.reshape(B, H, N)


# speedup vs baseline: 1.6456x; 1.0267x over previous
import jax
import jax.numpy as jnp
from jax.experimental import pallas as pl
from jax.experimental.pallas import tpu as pltpu


def _softmax_block_kernel(x_ref, lp_ref, p_ref):
    x = x_ref[...]
    m = jnp.max(x, axis=-1, keepdims=True)
    e = jnp.exp(x - m)
    s = jnp.sum(e, axis=-1, keepdims=True)
    lp_ref[...] = x - (m + jnp.log(s))
    p_ref[...] = e * (1.0 / s)


def kernel(x):
    B, H, N = x.shape
    rows = B * H
    xf = x.reshape(rows, N)
    R = 64
    grid = (rows // R,)
    lp, p = pl.pallas_call(
        _softmax_block_kernel,
        grid=grid,
        in_specs=[pl.BlockSpec((R, N), lambda i: (i, 0))],
        out_specs=[
            pl.BlockSpec((R, N), lambda i: (i, 0)),
            pl.BlockSpec((R, N), lambda i: (i, 0)),
        ],
        out_shape=[
            jax.ShapeDtypeStruct((rows, N), x.dtype),
            jax.ShapeDtypeStruct((rows, N), x.dtype),
        ],
        compiler_params=pltpu.CompilerParams(
            dimension_semantics=("arbitrary",),
        ),
    )(xf)
    return lp.reshape(B, H, N), p.reshape(B, H, N)
